# Initial kernel scaffold; baseline (speedup 1.0000x reference)
import jax
import jax.numpy as jnp
from jax.experimental import pallas as pl


def kernel(x, weights, indices):
    raise NotImplementedError("write your pallas kernel here")


# speedup vs baseline: 1.0871x; 1.0871x over previous
import functools

import jax
import jax.numpy as jnp
from jax import lax
from jax.experimental import pallas as pl
from jax.experimental.pallas import tpu as pltpu
from jax.experimental.pallas import tpu_sc as plsc

IN_DIM = 4096
OUT_DIM = 8192
BATCH = 1024
NUM_FUNCTIONS = 16

B_LOC = 16
NB = 512
N_WORKERS = 32
CHUNKS_PER_WORKER = BATCH // (B_LOC * N_WORKERS)

_FOLD = (
    (0.0, 0.0, 0.0, 0.0),
    (0.0, 0.0, 0.0, 1.0),
    (0.0, 1.0, 0.0, -1.0),
    (0.0, 1.0, 0.0, 0.0),
    (0.0, 0.0, 1.0, -1.0),
    (0.0, 0.0, 1.0, 0.0),
    (0.0, 1.0, 1.0, -2.0),
    (0.0, 1.0, 1.0, -1.0),
    (1.0, -1.0, -1.0, 1.0),
    (1.0, -1.0, -1.0, 2.0),
    (1.0, 0.0, -1.0, 0.0),
    (1.0, 0.0, -1.0, 1.0),
    (1.0, -1.0, 0.0, 0.0),
    (1.0, -1.0, 0.0, 1.0),
    (1.0, 0.0, 0.0, -1.0),
    (1.0, 0.0, 0.0, 0.0),
)


def _coef_body(wt_ref, c_ref):
    w = wt_ref[...]
    m = jnp.max(w, axis=0, keepdims=True)
    e = jnp.exp(w - m)
    s = jnp.sum(e, axis=0, keepdims=True)
    p = e / s
    rows = []
    for k in range(4):
        acc = jnp.zeros_like(p[0:1, :])
        for i in range(NUM_FUNCTIONS):
            coef = _FOLD[i][k]
            if coef:
                acc = acc + coef * p[i:i + 1, :]
        rows.append(acc)
    c_ref[...] = jnp.concatenate(rows, axis=0)


_coef_call = pl.pallas_call(
    _coef_body,
    out_shape=jax.ShapeDtypeStruct((4, OUT_DIM), jnp.float32),
)


def _sc_body(x_hbm, idx_hbm, c_hbm, out_hbm, x_loc, idx_loc, c_loc, out_loc):
    wid = lax.axis_index("s") * 2 + lax.axis_index("c")
    pltpu.sync_copy(idx_hbm, idx_loc)
    pltpu.sync_copy(c_hbm, c_loc)
    for ci in range(CHUNKS_PER_WORKER):
        chunk = wid * CHUNKS_PER_WORKER + ci
        row0 = chunk * B_LOC
        pltpu.sync_copy(x_hbm.at[pl.ds(row0, B_LOC), :], x_loc)

        def blk_body(bi, _, row0=row0):
            col0 = bi * NB

            def jv_body(jj, _, col0=col0):
                j0 = col0 + jj * 16
                i0 = idx_loc[0, pl.ds(j0, 16)]
                i1 = idx_loc[1, pl.ds(j0, 16)]
                c0 = c_loc[0, pl.ds(j0, 16)]
                c1 = c_loc[1, pl.ds(j0, 16)]
                c2 = c_loc[2, pl.ds(j0, 16)]
                c3 = c_loc[3, pl.ds(j0, 16)]
                for t in range(B_LOC):
                    tv = jnp.full((16,), t, jnp.int32)
                    a = plsc.load_gather(x_loc, [tv, i0])
                    b = plsc.load_gather(x_loc, [tv, i1])
                    o = c0 + c1 * a + c2 * b + c3 * (a * b)
                    out_loc[t, pl.ds(jj * 16, 16)] = o
                return 0

            lax.fori_loop(0, NB // 16, jv_body, 0)
            pltpu.sync_copy(out_loc,
                            out_hbm.at[pl.ds(row0, B_LOC), pl.ds(col0, NB)])
            return 0

        lax.fori_loop(0, OUT_DIM // NB, blk_body, 0)


@functools.cache
def _sc_call():
    return functools.partial(
        pl.kernel,
        out_type=jax.ShapeDtypeStruct((BATCH, OUT_DIM), jnp.float32),
        mesh=plsc.VectorSubcoreMesh(core_axis_name="c", subcore_axis_name="s"),
        compiler_params=pltpu.CompilerParams(needs_layout_passes=False),
        scratch_types=[
            pltpu.VMEM((B_LOC, IN_DIM), jnp.float32),
            pltpu.VMEM((2, OUT_DIM), jnp.int32),
            pltpu.VMEM((4, OUT_DIM), jnp.float32),
            pltpu.VMEM((B_LOC, NB), jnp.float32),
        ],
    )(_sc_body)


@jax.jit
def kernel(x, weights, indices):
    idx32 = indices.astype(jnp.int32)
    c = _coef_call(weights.T)
    return _sc_call()(x, idx32, c)


# speedup vs baseline: 1.7578x; 1.6170x over previous
import functools

import jax
import jax.numpy as jnp
from jax import lax
from jax.experimental import pallas as pl
from jax.experimental.pallas import tpu as pltpu
from jax.experimental.pallas import tpu_sc as plsc

IN_DIM = 4096
OUT_DIM = 8192
BATCH = 1024
NUM_FUNCTIONS = 16

B_LOC = 16
NB = 512
N_WORKERS = 32
CHUNKS_PER_WORKER = BATCH // (B_LOC * N_WORKERS)

_FOLD = (
    (0.0, 0.0, 0.0, 0.0),
    (0.0, 0.0, 0.0, 1.0),
    (0.0, 1.0, 0.0, -1.0),
    (0.0, 1.0, 0.0, 0.0),
    (0.0, 0.0, 1.0, -1.0),
    (0.0, 0.0, 1.0, 0.0),
    (0.0, 1.0, 1.0, -2.0),
    (0.0, 1.0, 1.0, -1.0),
    (1.0, -1.0, -1.0, 1.0),
    (1.0, -1.0, -1.0, 2.0),
    (1.0, 0.0, -1.0, 0.0),
    (1.0, 0.0, -1.0, 1.0),
    (1.0, -1.0, 0.0, 0.0),
    (1.0, -1.0, 0.0, 1.0),
    (1.0, 0.0, 0.0, -1.0),
    (1.0, 0.0, 0.0, 0.0),
)


def _coef_body(wt_ref, c_ref):
    w = wt_ref[...]
    m = jnp.max(w, axis=0, keepdims=True)
    e = jnp.exp(w - m)
    s = jnp.sum(e, axis=0, keepdims=True)
    p = e / s
    rows = []
    for k in range(4):
        acc = jnp.zeros_like(p[0:1, :])
        for i in range(NUM_FUNCTIONS):
            coef = _FOLD[i][k]
            if coef:
                acc = acc + coef * p[i:i + 1, :]
        rows.append(acc)
    c_ref[...] = jnp.concatenate(rows, axis=0)


_coef_call = pl.pallas_call(
    _coef_body,
    out_shape=jax.ShapeDtypeStruct((4, OUT_DIM), jnp.float32),
)


def _sc_body(x_hbm, idx_hbm, c_hbm, out_hbm, x_loc, idx_loc, c_loc, out_loc):
    wid = lax.axis_index("s") * 2 + lax.axis_index("c")
    pltpu.sync_copy(idx_hbm, idx_loc)
    pltpu.sync_copy(c_hbm, c_loc)
    for ci in range(CHUNKS_PER_WORKER):
        chunk = wid * CHUNKS_PER_WORKER + ci
        row0 = chunk * B_LOC
        pltpu.sync_copy(x_hbm.at[pl.ds(row0, B_LOC), :], x_loc)

        def blk_body(bi, _, row0=row0):
            col0 = bi * NB

            @plsc.parallel_loop(0, NB // 16, unroll=2)
            def jv_body(jj, col0=col0):
                j0 = col0 + jj * 16
                i0 = idx_loc[0, pl.ds(j0, 16)]
                i1 = idx_loc[1, pl.ds(j0, 16)]
                c0 = c_loc[0, pl.ds(j0, 16)]
                c1 = c_loc[1, pl.ds(j0, 16)]
                c2 = c_loc[2, pl.ds(j0, 16)]
                c3 = c_loc[3, pl.ds(j0, 16)]
                for t in range(B_LOC):
                    tv = jnp.full((16,), t, jnp.int32)
                    a = plsc.load_gather(x_loc, [tv, i0])
                    b = plsc.load_gather(x_loc, [tv, i1])
                    o = c0 + c1 * a + c2 * b + c3 * (a * b)
                    out_loc[t, pl.ds(jj * 16, 16)] = o
            pltpu.sync_copy(out_loc,
                            out_hbm.at[pl.ds(row0, B_LOC), pl.ds(col0, NB)])
            return 0

        lax.fori_loop(0, OUT_DIM // NB, blk_body, 0)


@functools.cache
def _sc_call():
    return functools.partial(
        pl.kernel,
        out_type=jax.ShapeDtypeStruct((BATCH, OUT_DIM), jnp.float32),
        mesh=plsc.VectorSubcoreMesh(core_axis_name="c", subcore_axis_name="s"),
        compiler_params=pltpu.CompilerParams(needs_layout_passes=False),
        scratch_types=[
            pltpu.VMEM((B_LOC, IN_DIM), jnp.float32),
            pltpu.VMEM((2, OUT_DIM), jnp.int32),
            pltpu.VMEM((4, OUT_DIM), jnp.float32),
            pltpu.VMEM((B_LOC, NB), jnp.float32),
        ],
    )(_sc_body)


@jax.jit
def kernel(x, weights, indices):
    idx32 = indices.astype(jnp.int32)
    c = _coef_call(weights.T)
    return _sc_call()(x, idx32, c)
